# baseline (speedup 1.0000x reference)
import functools

import jax
import jax.numpy as jnp
from jax import lax
from jax.experimental import pallas as pl
from jax.experimental.pallas import tpu as pltpu
from jax.experimental.pallas import tpu_sc as plsc

_ROW_DEPTH = 30
_LANES = 16


@functools.cache
def _build(batch: int, n_seq: int):
    assert n_seq == 32768, "kernel is specialized to the pipeline shape"
    info = plsc.get_sparse_core_info()
    n_workers = info.num_cores * info.num_subcores
    assert batch % n_workers == 0
    rows_per_worker = batch // n_workers

    n_full = 1088
    chunks = [(0, 8, 0, 8744)] + [
        (8 + 2 * t, 10 + 2 * t, 8744 + 2184 * t, 2184) for t in range(11)
    ]
    buf_words = 8744 + _LANES

    mesh = plsc.VectorSubcoreMesh(core_axis_name="c", subcore_axis_name="s")

    @functools.partial(
        pl.kernel,
        mesh=mesh,
        out_type=jax.ShapeDtypeStruct((batch, n_seq), jnp.float32),
        scratch_types=[
            pltpu.VMEM((n_seq,), jnp.float32),
            pltpu.VMEM((n_seq,), jnp.float32),
            pltpu.VMEM((buf_words,), jnp.float32),
            pltpu.VMEM((buf_words,), jnp.float32),
            pltpu.SemaphoreType.DMA,
            pltpu.SemaphoreType.DMA,
            pltpu.SemaphoreType.DMA,
            pltpu.SemaphoreType.DMA,
        ],
        compiler_params=pltpu.CompilerParams(
            needs_layout_passes=False, use_tc_tiling_on_sc=False
        ),
    )
    def interleave(in_hbm, out_hbm, in_v0, in_v1, out_v0, out_v1,
                   sem_i0, sem_i1, sem_o0, sem_o1):
        wid = lax.axis_index("s") * info.num_cores + lax.axis_index("c")
        row0 = wid * rows_per_worker
        in_bufs, sem_ins = [in_v0, in_v1], [sem_i0, sem_i1]
        out_bufs, sem_outs = [out_v0, out_v1], [sem_o0, sem_o1]
        v30 = lax.iota(jnp.int32, _LANES) * _ROW_DEPTH

        h_in = [None, None]
        h_out = [None, None]
        h_in[0] = pltpu.async_copy(in_hbm.at[row0], in_bufs[0], sem_ins[0])

        for j in range(rows_per_worker):
            jb = j % 2
            h_in[jb].wait()
            if j + 1 < rows_per_worker:
                nb = (j + 1) % 2
                h_in[nb] = pltpu.async_copy(
                    in_hbm.at[row0 + j + 1], in_bufs[nb], sem_ins[nb]
                )
            src = in_bufs[jb]
            for t, (c0, c1, base, words) in enumerate(chunks):
                b = (j * len(chunks) + t) % 2
                if h_out[b] is not None:
                    h_out[b].wait()
                dst = out_bufs[b]

                def col_body(c, carry, _dst=dst, _base=base):
                    off_c = jnp.where(
                        c < 8, c * 1093, 8744 + (c - 8) * 1092
                    )
                    local = off_c - _base

                    @plsc.parallel_loop(0, n_full, step=_LANES, unroll=8)
                    def gather16(i):
                        idx = v30 + (i * _ROW_DEPTH + c)
                        _dst[pl.ds(local + i, _LANES)] = plsc.load_gather(
                            src, [idx]
                        )

                    idx_t = jnp.minimum(
                        v30 + (n_full * _ROW_DEPTH + c), n_seq - 1
                    )
                    _dst[pl.ds(local + n_full, _LANES)] = plsc.load_gather(
                        src, [idx_t]
                    )
                    return carry

                lax.fori_loop(c0, c1, col_body, 0)
                h_out[b] = pltpu.async_copy(
                    dst.at[pl.ds(0, words)],
                    out_hbm.at[row0 + j, pl.ds(base, words)],
                    sem_outs[b],
                )
        h_out[0].wait()
        h_out[1].wait()

    return interleave


def kernel(inputs):
    batch, n_seq = inputs.shape
    return _build(batch, n_seq)(inputs)


# speedup vs baseline: 1.8098x; 1.8098x over previous
import functools

import jax
import jax.numpy as jnp
from jax import lax
from jax.experimental import pallas as pl
from jax.experimental.pallas import tpu as pltpu
from jax.experimental.pallas import tpu_sc as plsc

_ROW_DEPTH = 30
_LANES = 16
_CHUNK = 8192


def _segments(n_seq: int, r_depth: int, chunk: int):
    import numpy as np

    n = int(np.ceil(n_seq / r_depth) * r_depth)
    nb = n // r_depth
    lens = [sum(1 for r in range(nb) if r * r_depth + c < n_seq)
            for c in range(r_depth)]
    offs = np.concatenate([[0], np.cumsum(lens)])
    per_chunk = []
    for k in range(n_seq // chunk):
        base, end = k * chunk, (k + 1) * chunk
        segs = []
        for c in range(r_depth):
            lo, hi = int(offs[c]), int(offs[c + 1])
            if hi <= base or lo >= end:
                continue
            r_lo = max(0, base - lo)
            r_hi = min(lens[c], end - lo)
            segs.append((c, r_lo, r_hi, lo + r_lo - base))
        per_chunk.append(segs)
    return per_chunk


@functools.cache
def _build(batch: int, n_seq: int):
    info = plsc.get_sparse_core_info()
    n_workers = info.num_cores * info.num_subcores
    assert batch % n_workers == 0
    assert n_seq % _CHUNK == 0
    rows_per_worker = batch // n_workers
    per_chunk = _segments(n_seq, _ROW_DEPTH, _CHUNK)
    n_chunks = len(per_chunk)

    mesh = plsc.VectorSubcoreMesh(core_axis_name="c", subcore_axis_name="s")

    @functools.partial(
        pl.kernel,
        mesh=mesh,
        out_type=jax.ShapeDtypeStruct((batch, n_seq), jnp.float32),
        scratch_types=[
            pltpu.VMEM((n_seq,), jnp.float32),
            pltpu.VMEM((n_seq,), jnp.float32),
            pltpu.VMEM((_CHUNK + _LANES,), jnp.float32),
            pltpu.VMEM((_CHUNK + _LANES,), jnp.float32),
            pltpu.SemaphoreType.DMA,
            pltpu.SemaphoreType.DMA,
            pltpu.SemaphoreType.DMA,
            pltpu.SemaphoreType.DMA,
        ],
        compiler_params=pltpu.CompilerParams(needs_layout_passes=False),
    )
    def interleave(in_hbm, out_hbm, in_v0, in_v1, out_v0, out_v1,
                   sem_i0, sem_i1, sem_o0, sem_o1):
        wid = lax.axis_index("s") * info.num_cores + lax.axis_index("c")
        row0 = wid * rows_per_worker
        in_bufs, sem_ins = [in_v0, in_v1], [sem_i0, sem_i1]
        out_bufs, sem_outs = [out_v0, out_v1], [sem_o0, sem_o1]
        v30 = lax.iota(jnp.int32, _LANES) * _ROW_DEPTH

        h_in = [None, None]
        h_out = [None, None]
        h_in[0] = pltpu.async_copy(in_hbm.at[row0], in_bufs[0], sem_ins[0])

        for j in range(rows_per_worker):
            jb = j % 2
            h_in[jb].wait()
            if j + 1 < rows_per_worker:
                nb2 = (j + 1) % 2
                h_in[nb2] = pltpu.async_copy(
                    in_hbm.at[row0 + j + 1], in_bufs[nb2], sem_ins[nb2]
                )
            src = in_bufs[jb]
            for k in range(n_chunks):
                b = (j * n_chunks + k) % 2
                if h_out[b] is not None:
                    h_out[b].wait()
                dst = out_bufs[b]
                for (c, r_lo, r_hi, local) in per_chunk[k]:
                    n = r_hi - r_lo
                    nf = n // _LANES * _LANES
                    if nf:
                        @plsc.parallel_loop(
                            0, nf, step=_LANES,
                            unroll=(8 if nf % (8 * _LANES) == 0 else 4),
                        )
                        def gather16(i, _c=c, _r=r_lo, _l=local, _d=dst):
                            idx = v30 + ((_r + i) * _ROW_DEPTH + _c)
                            _d[pl.ds(_l + i, _LANES)] = plsc.load_gather(
                                src, [idx]
                            )
                    if n % _LANES:
                        idx_t = jnp.minimum(
                            v30 + ((r_lo + nf) * _ROW_DEPTH + c),
                            n_seq - 1,
                        )
                        dst[pl.ds(local + nf, _LANES)] = plsc.load_gather(
                            src, [idx_t]
                        )
                h_out[b] = pltpu.async_copy(
                    dst.at[pl.ds(0, _CHUNK)],
                    out_hbm.at[row0 + j, pl.ds(k * _CHUNK, _CHUNK)],
                    sem_outs[b],
                )
        h_out[0].wait()
        h_out[1].wait()

    return interleave


def kernel(inputs):
    batch, n_seq = inputs.shape
    return _build(batch, n_seq)(inputs)


# speedup vs baseline: 2.1025x; 1.1618x over previous
import functools

import numpy as np
import jax
import jax.numpy as jnp
from jax import lax
from jax.experimental import pallas as pl
from jax.experimental.pallas import tpu as pltpu
from jax.experimental.pallas import tpu_sc as plsc

_ROW_DEPTH = 30
_LANES = 16
_OUT_CHUNKS = 4
_NBUF_OUT = 3


def _packed_perm(n_seq: int, r_depth: int) -> np.ndarray:
    n = int(np.ceil(n_seq / r_depth) * r_depth)
    nb_rows = n // r_depth
    ind = np.arange(n, dtype=np.int32)
    perm = ind.reshape(nb_rows, r_depth).T.reshape(-1)
    perm = perm[perm < n_seq].astype(np.int32)
    p = perm.reshape(-1, 2, _LANES)
    return (p[:, 0, :] | (p[:, 1, :] << 16)).astype(np.int32).reshape(-1)


@functools.cache
def _build(batch: int, n_seq: int):
    info = plsc.get_sparse_core_info()
    n_workers = info.num_cores * info.num_subcores
    assert batch % n_workers == 0
    assert n_seq % (2 * _LANES * _OUT_CHUNKS) == 0
    rows_per_worker = batch // n_workers
    chunk = n_seq // _OUT_CHUNKS
    pairs_per_chunk = chunk // (2 * _LANES)

    mesh = plsc.VectorSubcoreMesh(core_axis_name="c", subcore_axis_name="s")

    @functools.partial(
        pl.kernel,
        mesh=mesh,
        out_type=jax.ShapeDtypeStruct((batch, n_seq), jnp.float32),
        scratch_types=[
            pltpu.VMEM((n_seq // 2,), jnp.int32),
            pltpu.VMEM((n_seq,), jnp.float32),
            pltpu.VMEM((n_seq,), jnp.float32),
            pltpu.VMEM((chunk,), jnp.float32),
            pltpu.VMEM((chunk,), jnp.float32),
            pltpu.VMEM((chunk,), jnp.float32),
            pltpu.SemaphoreType.DMA,
            pltpu.SemaphoreType.DMA,
            pltpu.SemaphoreType.DMA,
            pltpu.SemaphoreType.DMA,
            pltpu.SemaphoreType.DMA,
        ],
        compiler_params=pltpu.CompilerParams(needs_layout_passes=False),
    )
    def interleave(in_hbm, pp_hbm, out_hbm, pp_v, in_v0, in_v1,
                   out_v0, out_v1, out_v2,
                   sem_i0, sem_i1, sem_o0, sem_o1, sem_o2):
        wid = lax.axis_index("s") * info.num_cores + lax.axis_index("c")
        row0 = wid * rows_per_worker
        in_bufs, sem_ins = [in_v0, in_v1], [sem_i0, sem_i1]
        out_bufs = [out_v0, out_v1, out_v2]
        sem_outs = [sem_o0, sem_o1, sem_o2]

        h_in = [None, None]
        h_out = [None] * _NBUF_OUT
        h_in[0] = pltpu.async_copy(in_hbm.at[row0], in_bufs[0], sem_ins[0])
        pltpu.sync_copy(pp_hbm, pp_v)

        for j in range(rows_per_worker):
            jb = j % 2
            h_in[jb].wait()
            if j + 1 < rows_per_worker:
                nb2 = (j + 1) % 2
                h_in[nb2] = pltpu.async_copy(
                    in_hbm.at[row0 + j + 1], in_bufs[nb2], sem_ins[nb2]
                )
            src = in_bufs[jb]
            for k in range(_OUT_CHUNKS):
                b = (j * _OUT_CHUNKS + k) % _NBUF_OUT
                if h_out[b] is not None:
                    h_out[b].wait()
                dst = out_bufs[b]
                pbase = k * chunk // 2

                @plsc.parallel_loop(0, pairs_per_chunk, unroll=8)
                def gather32(q, _dst=dst, _src=src, _pb=pbase):
                    v = pp_v[pl.ds(_pb + q * _LANES, _LANES)]
                    lo = v & jnp.int32(0xFFFF)
                    hi = lax.shift_right_logical(v, jnp.int32(16))
                    _dst[pl.ds(q * 2 * _LANES, _LANES)] = plsc.load_gather(
                        _src, [lo]
                    )
                    _dst[pl.ds(q * 2 * _LANES + _LANES, _LANES)] = (
                        plsc.load_gather(_src, [hi])
                    )

                h_out[b] = pltpu.async_copy(
                    dst,
                    out_hbm.at[row0 + j, pl.ds(k * chunk, chunk)],
                    sem_outs[b],
                )
        for b in range(_NBUF_OUT):
            h_out[b].wait()

    return interleave


def kernel(inputs):
    batch, n_seq = inputs.shape
    packed = jnp.asarray(_packed_perm(n_seq, _ROW_DEPTH))
    return _build(batch, n_seq)(inputs, packed)


# speedup vs baseline: 2.1103x; 1.0037x over previous
import functools

import numpy as np
import jax
import jax.numpy as jnp
from jax import lax
from jax.experimental import pallas as pl
from jax.experimental.pallas import tpu as pltpu
from jax.experimental.pallas import tpu_sc as plsc

_ROW_DEPTH = 30
_LANES = 16
_OUT_CHUNKS = 4
_NBUF_OUT = 4


def _packed_perm(n_seq: int, r_depth: int) -> np.ndarray:
    n = int(np.ceil(n_seq / r_depth) * r_depth)
    nb_rows = n // r_depth
    ind = np.arange(n, dtype=np.int32)
    perm = ind.reshape(nb_rows, r_depth).T.reshape(-1)
    perm = perm[perm < n_seq].astype(np.int32)
    p = perm.reshape(-1, 2, _LANES)
    return (p[:, 0, :] | (p[:, 1, :] << 16)).astype(np.int32).reshape(-1)


@functools.cache
def _build(batch: int, n_seq: int):
    info = plsc.get_sparse_core_info()
    n_workers = info.num_cores * info.num_subcores
    assert batch % n_workers == 0
    assert n_seq % (2 * _LANES * _OUT_CHUNKS) == 0
    rows_per_worker = batch // n_workers
    chunk = n_seq // _OUT_CHUNKS
    pairs_per_chunk = chunk // (2 * _LANES)

    mesh = plsc.VectorSubcoreMesh(core_axis_name="c", subcore_axis_name="s")

    @functools.partial(
        pl.kernel,
        mesh=mesh,
        out_type=jax.ShapeDtypeStruct((batch, n_seq), jnp.float32),
        scratch_types=[
            pltpu.VMEM((n_seq // 2,), jnp.int32),
            pltpu.VMEM((n_seq,), jnp.float32),
            pltpu.VMEM((n_seq,), jnp.float32),
            pltpu.VMEM((chunk,), jnp.float32),
            pltpu.VMEM((chunk,), jnp.float32),
            pltpu.VMEM((chunk,), jnp.float32),
            pltpu.VMEM((chunk,), jnp.float32),
            pltpu.SemaphoreType.DMA,
            pltpu.SemaphoreType.DMA,
            pltpu.SemaphoreType.DMA,
            pltpu.SemaphoreType.DMA,
            pltpu.SemaphoreType.DMA,
            pltpu.SemaphoreType.DMA,
        ],
        compiler_params=pltpu.CompilerParams(needs_layout_passes=False),
    )
    def interleave(in_hbm, pp_hbm, out_hbm, pp_v, in_v0, in_v1,
                   out_v0, out_v1, out_v2, out_v3,
                   sem_i0, sem_i1, sem_o0, sem_o1, sem_o2, sem_o3):
        wid = lax.axis_index("s") * info.num_cores + lax.axis_index("c")
        row0 = wid * rows_per_worker
        in_bufs, sem_ins = [in_v0, in_v1], [sem_i0, sem_i1]
        out_bufs = [out_v0, out_v1, out_v2, out_v3]
        sem_outs = [sem_o0, sem_o1, sem_o2, sem_o3]

        h_in = [None, None]
        h_out = [None] * _NBUF_OUT
        h_in[0] = pltpu.async_copy(in_hbm.at[row0], in_bufs[0], sem_ins[0])
        pltpu.sync_copy(pp_hbm, pp_v)

        for j in range(rows_per_worker):
            jb = j % 2
            h_in[jb].wait()
            if j + 1 < rows_per_worker:
                nb2 = (j + 1) % 2
                h_in[nb2] = pltpu.async_copy(
                    in_hbm.at[row0 + j + 1], in_bufs[nb2], sem_ins[nb2]
                )
            src = in_bufs[jb]
            for k in range(_OUT_CHUNKS):
                b = (j * _OUT_CHUNKS + k) % _NBUF_OUT
                if h_out[b] is not None:
                    h_out[b].wait()
                dst = out_bufs[b]
                pbase = k * chunk // 2

                @plsc.parallel_loop(0, pairs_per_chunk, unroll=8)
                def gather32(q, _dst=dst, _src=src, _pb=pbase):
                    v = pp_v[pl.ds(_pb + q * _LANES, _LANES)]
                    lo = v & jnp.int32(0xFFFF)
                    hi = lax.shift_right_logical(v, jnp.int32(16))
                    _dst[pl.ds(q * 2 * _LANES, _LANES)] = plsc.load_gather(
                        _src, [lo]
                    )
                    _dst[pl.ds(q * 2 * _LANES + _LANES, _LANES)] = (
                        plsc.load_gather(_src, [hi])
                    )

                h_out[b] = pltpu.async_copy(
                    dst,
                    out_hbm.at[row0 + j, pl.ds(k * chunk, chunk)],
                    sem_outs[b],
                )
        for b in range(_NBUF_OUT):
            h_out[b].wait()

    return interleave


def kernel(inputs):
    batch, n_seq = inputs.shape
    packed = jnp.asarray(_packed_perm(n_seq, _ROW_DEPTH))
    return _build(batch, n_seq)(inputs, packed)
